# baseline (speedup 1.0000x reference)
import functools

import jax
import jax.numpy as jnp
from jax import lax
from jax.experimental import pallas as pl
from jax.experimental.pallas import tpu as pltpu
from jax.experimental.pallas import tpu_sc as plsc

NC = 2
NS = 16
NW = NC * NS
LANES = 16

BATCH = 4096
HIST = 50
EMBED_DIM = 64
NSEG = EMBED_DIM // LANES
TOTAL = BATCH * HIST
PER_W = TOTAL // NW
CHUNK = 128
NCHUNK = PER_W // CHUNK
NBUF = 5
NSUPER = NCHUNK // NBUF


def _sc_body(table_hbm, idx_hbm, scale_hbm, out_hbm,
             idx_v, scale_v,
             a0, a1, a2, a3, a4, b0, b1, b2, b3, b4,
             g0, g1, g2, g3, g4, s0, s1, s2, s3, s4, idx_sem):
    wid = lax.axis_index("s") * NC + lax.axis_index("c")

    pltpu.async_copy(idx_hbm.at[wid], idx_v, idx_sem)
    pltpu.sync_copy(scale_hbm, scale_v)
    svecs = [scale_v[pl.ds(c * LANES, LANES)] for c in range(NSEG)]
    pltpu.make_async_copy(idx_hbm.at[wid], idx_v, idx_sem).wait()

    A = (a0, a1, a2, a3, a4)
    B = (b0, b1, b2, b3, b4)
    GS = (g0, g1, g2, g3, g4)
    SS = (s0, s1, s2, s3, s4)

    def g_start(j, b):
        pltpu.async_copy(table_hbm.at[idx_v.at[j]], A[b], GS[b])

    def g_wait(b):
        pltpu.make_async_copy(table_hbm.at[idx_v.at[0]], A[b], GS[b]).wait()

    def s_start(j, b):
        pltpu.async_copy(B[b], out_hbm.at[wid, j], SS[b])

    def s_wait(b):
        pltpu.make_async_copy(B[b], out_hbm.at[0, 0], SS[b]).wait()

    for b in range(NBUF):
        g_start(b, b)

    def superstep(s, carry):
        for b in range(NBUF):
            j = s * NBUF + b
            g_wait(b)

            @pl.when(s >= 1)
            def _():
                s_wait(b)

            def row_body(r, c, b=b):
                for seg in range(NSEG):
                    B[b][r, pl.ds(seg * LANES, LANES)] = (
                        A[b][r, pl.ds(seg * LANES, LANES)] * svecs[seg]
                    )
                return c

            lax.fori_loop(0, CHUNK, row_body, 0, unroll=4)

            @pl.when(s < NSUPER - 1)
            def _():
                g_start(j + NBUF, b)

            s_start(j, b)
        return carry

    lax.fori_loop(0, NSUPER, superstep, 0)

    for b in range(NBUF):
        s_wait(b)


@jax.jit
def _run(table, idx3, scale):
    mesh = plsc.VectorSubcoreMesh(
        core_axis_name="c", subcore_axis_name="s", num_cores=NC, num_subcores=NS
    )
    f = pl.kernel(
        _sc_body,
        out_type=jax.ShapeDtypeStruct((NW, NCHUNK, CHUNK, EMBED_DIM), jnp.float32),
        mesh=mesh,
        compiler_params=pltpu.CompilerParams(use_tc_tiling_on_sc=False),
        scratch_types=[
            pltpu.VMEM((NCHUNK, CHUNK), jnp.int32),
            pltpu.VMEM((EMBED_DIM,), jnp.float32),
        ]
        + [pltpu.VMEM((CHUNK, EMBED_DIM), jnp.float32) for _ in range(2 * NBUF)]
        + [pltpu.SemaphoreType.DMA for _ in range(2 * NBUF + 1)],
    )
    return f(table, idx3, scale)


def kernel(x, embed_weight, p_adic_scale):
    idx3 = x.astype(jnp.int32).reshape(NW, NCHUNK, CHUNK)
    out = _run(embed_weight, idx3, p_adic_scale)
    return out.reshape(BATCH, HIST, EMBED_DIM)


# speedup vs baseline: 1.2257x; 1.2257x over previous
import functools

import jax
import jax.numpy as jnp
from jax import lax
from jax.experimental import pallas as pl
from jax.experimental.pallas import tpu as pltpu
from jax.experimental.pallas import tpu_sc as plsc

NC = 2
NS = 16
NW = NC * NS
LANES = 16

BATCH = 4096
HIST = 50
EMBED_DIM = 64
BBLK = BATCH // NW
NBUF = 4


def _sc_body(table_hbm, idx_hbm, inter_hbm, idx_v, b0, b1, b2, b3,
             g0, g1, g2, g3, s0, s1, s2, s3, idx_sem):
    wid = lax.axis_index("s") * NC + lax.axis_index("c")
    col0 = wid * BBLK

    pltpu.sync_copy(idx_hbm.at[wid], idx_v)

    B = (b0, b1, b2, b3)
    GS = (g0, g1, g2, g3)
    SS = (s0, s1, s2, s3)

    def g_start(h, b):
        pltpu.async_copy(table_hbm.at[idx_v.at[h]], B[b], GS[b])

    def g_wait(b):
        pltpu.make_async_copy(table_hbm.at[idx_v.at[0]], B[b], GS[b]).wait()

    def s_start(h, b):
        pltpu.async_copy(B[b], inter_hbm.at[h, pl.ds(col0, BBLK)], SS[b])

    def s_wait(b):
        pltpu.make_async_copy(B[b], inter_hbm.at[0, pl.ds(0, BBLK)], SS[b]).wait()

    g_start(0, 0)
    g_start(1, 1)

    def superstep(s, carry):
        for u in range(NBUF):
            j = s * NBUF + u
            b = u
            bn = (u + 2) % NBUF
            g_wait(b)
            s_start(j, b)

            @pl.when(s * NBUF + u >= 2)
            def _():
                s_wait(bn)

            @pl.when(s * NBUF + u + 2 < HIST)
            def _():
                g_start(j + 2, bn)
        return carry

    lax.fori_loop(0, HIST // NBUF, superstep, 0)

    for j in (48, 49):
        b = j % NBUF
        bn = (j + 2) % NBUF
        g_wait(b)
        s_start(j, b)
        s_wait(bn)

    s_wait(0)
    s_wait(1)


def _tc_body(in_ref, scale_ref, out_ref):
    scale2 = jnp.concatenate([scale_ref[...], scale_ref[...]])
    fi = lax.broadcasted_iota(jnp.int32, (EMBED_DIM, 2 * EMBED_DIM), 0)
    bi = lax.broadcasted_iota(jnp.int32, (EMBED_DIM, 2 * EMBED_DIM), 1)
    sel = (fi == bi // 2).astype(jnp.float32)
    parity = lax.broadcasted_iota(jnp.int32, (EMBED_DIM, 2 * EMBED_DIM), 1) % 2

    for g in range(8):
        xg = in_ref[pl.ds(EMBED_DIM * g, EMBED_DIM), :]
        xs = xg * scale2[None, :]
        r = lax.dot_general(
            xs, sel, (((0,), (0,)), ((), ())),
            preferred_element_type=jnp.float32,
        )
        og = jnp.where(parity == 0, r[0:EMBED_DIM, :], r[EMBED_DIM:, :])
        out_ref[0, :, pl.ds(2 * EMBED_DIM * g, 2 * EMBED_DIM)] = og


@jax.jit
def _run(table, idx3, scale):
    mesh = plsc.VectorSubcoreMesh(
        core_axis_name="c", subcore_axis_name="s", num_cores=NC, num_subcores=NS
    )
    sc = pl.kernel(
        _sc_body,
        out_type=jax.ShapeDtypeStruct((HIST, BATCH, EMBED_DIM), jnp.float32),
        mesh=mesh,
        compiler_params=pltpu.CompilerParams(use_tc_tiling_on_sc=False),
        scratch_types=[
            pltpu.VMEM((HIST, BBLK), jnp.int32),
        ]
        + [pltpu.VMEM((BBLK, EMBED_DIM), jnp.float32) for _ in range(NBUF)]
        + [pltpu.SemaphoreType.DMA for _ in range(2 * NBUF)]
        + [pltpu.SemaphoreType.DMA],
    )
    inter = sc(table, idx3)
    interf = inter.reshape(HIST * BATCH * EMBED_DIM // 128, 128)

    out_t = pl.pallas_call(
        _tc_body,
        out_shape=jax.ShapeDtypeStruct((HIST, EMBED_DIM, BATCH), jnp.float32),
        grid=(HIST, 4),
        in_specs=[
            pl.BlockSpec((512, 128), lambda h, w: (4 * h + w, 0)),
            pl.BlockSpec((EMBED_DIM,), lambda h, w: (0,)),
        ],
        out_specs=pl.BlockSpec((1, EMBED_DIM, BATCH // 4), lambda h, w: (h, 0, w)),
    )(interf, scale)

    return out_t.transpose(2, 0, 1)


def kernel(x, embed_weight, p_adic_scale):
    idx3 = x.astype(jnp.int32).reshape(NW, BBLK, HIST).swapaxes(1, 2)
    return _run(embed_weight, idx3, p_adic_scale)


# speedup vs baseline: 1.8004x; 1.4689x over previous
import functools

import jax
import jax.numpy as jnp
from jax import lax
from jax.experimental import pallas as pl
from jax.experimental.pallas import tpu as pltpu
from jax.experimental.pallas import tpu_sc as plsc

NC = 2
NS = 16
NW = NC * NS
LANES = 16

BATCH = 4096
HIST = 50
EMBED_DIM = 64
BBLK = BATCH // NW
NBUF = 4


def _sc_body(table_hbm, idx_hbm, inter_hbm, idx_v, b0, b1, b2, b3,
             g0, g1, g2, g3, s0, s1, s2, s3, idx_sem):
    wid = lax.axis_index("s") * NC + lax.axis_index("c")
    col0 = wid * BBLK

    pltpu.sync_copy(idx_hbm.at[wid], idx_v)

    B = (b0, b1, b2, b3)
    GS = (g0, g1, g2, g3)
    SS = (s0, s1, s2, s3)

    def g_start(h, b):
        pltpu.async_copy(table_hbm.at[idx_v.at[h]], B[b], GS[b])

    def g_wait(b):
        pltpu.make_async_copy(table_hbm.at[idx_v.at[0]], B[b], GS[b]).wait()

    def s_start(h, b):
        pltpu.async_copy(B[b], inter_hbm.at[h, pl.ds(col0, BBLK)], SS[b])

    def s_wait(b):
        pltpu.make_async_copy(B[b], inter_hbm.at[0, pl.ds(0, BBLK)], SS[b]).wait()

    g_start(0, 0)
    g_start(1, 1)

    def superstep(s, carry):
        for u in range(NBUF):
            j = s * NBUF + u
            b = u
            bn = (u + 2) % NBUF
            g_wait(b)
            s_start(j, b)

            @pl.when(s * NBUF + u >= 2)
            def _():
                s_wait(bn)

            @pl.when(s * NBUF + u + 2 < HIST)
            def _():
                g_start(j + 2, bn)
        return carry

    lax.fori_loop(0, HIST // NBUF, superstep, 0)

    for j in (48, 49):
        b = j % NBUF
        bn = (j + 2) % NBUF
        g_wait(b)
        s_start(j, b)
        s_wait(bn)

    s_wait(0)
    s_wait(1)


def _tc_body(in_ref, scale_ref, out_ref):
    scale2 = jnp.concatenate([scale_ref[...], scale_ref[...]])
    fi = lax.broadcasted_iota(jnp.int32, (EMBED_DIM, 2 * EMBED_DIM), 0)
    bi = lax.broadcasted_iota(jnp.int32, (EMBED_DIM, 2 * EMBED_DIM), 1)
    sel = (fi == bi // 2).astype(jnp.float32)
    parity = bi % 2

    for g in range(32):
        xg = in_ref[pl.ds(EMBED_DIM * g, EMBED_DIM), :]
        xs = xg * scale2[None, :]
        r = lax.dot_general(
            xs, sel, (((0,), (0,)), ((), ())),
            preferred_element_type=jnp.float32,
        )
        og = jnp.where(parity == 0, r[0:EMBED_DIM, :], r[EMBED_DIM:, :])
        out_ref[0, :, pl.ds(2 * EMBED_DIM * g, 2 * EMBED_DIM)] = og


@jax.jit
def _run(table, idx3, scale):
    mesh = plsc.VectorSubcoreMesh(
        core_axis_name="c", subcore_axis_name="s", num_cores=NC, num_subcores=NS
    )
    sc = pl.kernel(
        _sc_body,
        out_type=jax.ShapeDtypeStruct((HIST, BATCH, EMBED_DIM), jnp.float32),
        mesh=mesh,
        compiler_params=pltpu.CompilerParams(use_tc_tiling_on_sc=False),
        scratch_types=[
            pltpu.VMEM((HIST, BBLK), jnp.int32),
        ]
        + [pltpu.VMEM((BBLK, EMBED_DIM), jnp.float32) for _ in range(NBUF)]
        + [pltpu.SemaphoreType.DMA for _ in range(2 * NBUF)]
        + [pltpu.SemaphoreType.DMA],
    )
    inter = sc(table, idx3)
    interf = inter.reshape(HIST * BATCH * EMBED_DIM // 128, 128)

    out_t = pl.pallas_call(
        _tc_body,
        out_shape=jax.ShapeDtypeStruct((HIST, EMBED_DIM, BATCH), jnp.float32),
        grid=(HIST,),
        in_specs=[
            pl.BlockSpec((2048, 128), lambda h: (h, 0)),
            pl.BlockSpec((EMBED_DIM,), lambda h: (0,)),
        ],
        out_specs=pl.BlockSpec((1, EMBED_DIM, BATCH), lambda h: (h, 0, 0)),
    )(interf, scale)

    return out_t.transpose(2, 0, 1)


def kernel(x, embed_weight, p_adic_scale):
    idx3 = x.astype(jnp.int32).reshape(NW, BBLK, HIST).swapaxes(1, 2)
    return _run(embed_weight, idx3, p_adic_scale)
